# Initial kernel scaffold; baseline (speedup 1.0000x reference)
import functools

import jax
import jax.numpy as jnp
from jax import lax
from jax.experimental import pallas as pl
from jax.experimental.pallas import tpu as pltpu
from jax.experimental.pallas import tpu_sc as plsc

N_NODES = 10000
D = 128
E_EDGES = 320000
L_LEAF = 5000

NC = 2
NS = 16
NW = NC * NS

NP = 10240
EP = 323584
EPT = EP // NW
NBLK = EPT // 128
LP = 8192
LPT = LP // NW
LBLK = LPT // 128
RPS = NP // NS

_MESH = plsc.VectorSubcoreMesh(core_axis_name="c", subcore_axis_name="s")

f32 = jnp.float32


@functools.partial(
    pl.kernel,
    out_type=jax.ShapeDtypeStruct((NC * NP, 16), f32),
    mesh=_MESH,
    scratch_types=[
        pltpu.VMEM((128, 16), f32),
        pltpu.VMEM((1, 128), jnp.int32),
        pltpu.VMEM_SHARED((NP, 16), f32),
    ],
)
def _deg_kernel(dst2_hbm, zeros_hbm, ones_hbm, out_hbm, ones_v, dst_v, acc_sh):
    cid = lax.axis_index("c")
    sid = lax.axis_index("s")
    wid = sid * NC + cid
    pltpu.sync_copy(zeros_hbm.at[pl.ds(sid * RPS, RPS)],
                    acc_sh.at[pl.ds(sid * RPS, RPS)])
    pltpu.sync_copy(ones_hbm, ones_v)
    plsc.subcore_barrier()

    @pl.loop(0, NBLK)
    def _(b):
        pltpu.sync_copy(dst2_hbm.at[pl.ds(wid * NBLK + b, 1)], dst_v)
        pltpu.sync_copy(ones_v, acc_sh.at[dst_v.at[0]], add=True)

    plsc.subcore_barrier()
    pltpu.sync_copy(acc_sh.at[pl.ds(sid * RPS, RPS)],
                    out_hbm.at[pl.ds(cid * NP + sid * RPS, RPS)])


@functools.partial(
    pl.kernel,
    out_type=jax.ShapeDtypeStruct((NC * NP, D), f32),
    mesh=_MESH,
    scratch_types=[
        pltpu.VMEM((128,), jnp.int32),
        pltpu.VMEM((1, 128), jnp.int32),
        pltpu.VMEM((128, D), f32),
        pltpu.VMEM_SHARED((NP, D), f32),
        pltpu.SemaphoreType.DMA,
    ],
)
def _segsum_kernel(h_hbm, src_hbm, dst2_hbm, zeros_hbm, out_hbm,
                   src_v, dst_v, rows_v, acc_sh, sem):
    cid = lax.axis_index("c")
    sid = lax.axis_index("s")
    wid = sid * NC + cid
    pltpu.sync_copy(zeros_hbm.at[pl.ds(sid * RPS, RPS)],
                    acc_sh.at[pl.ds(sid * RPS, RPS)])
    plsc.subcore_barrier()

    @pl.loop(0, NBLK)
    def _(b):
        base = wid * EPT + b * 128
        pltpu.sync_copy(src_hbm.at[pl.ds(base, 128)], src_v)
        pltpu.async_copy(h_hbm.at[src_v], rows_v, sem).wait()
        pltpu.sync_copy(dst2_hbm.at[pl.ds(wid * NBLK + b, 1)], dst_v)
        pltpu.sync_copy(rows_v, acc_sh.at[dst_v.at[0]], add=True)

    plsc.subcore_barrier()
    pltpu.sync_copy(acc_sh.at[pl.ds(sid * RPS, RPS)],
                    out_hbm.at[pl.ds(cid * NP + sid * RPS, RPS)])


@functools.partial(
    pl.kernel,
    out_type=jax.ShapeDtypeStruct((LP, D), f32),
    mesh=_MESH,
    scratch_types=[
        pltpu.VMEM((128,), jnp.int32),
        pltpu.VMEM((128, D), f32),
        pltpu.SemaphoreType.DMA,
    ],
)
def _leaf_gather_kernel(h_hbm, leaf_hbm, out_hbm, idx_v, rows_v, sem):
    cid = lax.axis_index("c")
    sid = lax.axis_index("s")
    wid = sid * NC + cid

    @pl.loop(0, LBLK)
    def _(b):
        base = wid * LPT + b * 128
        pltpu.sync_copy(leaf_hbm.at[pl.ds(base, 128)], idx_v)
        pltpu.async_copy(h_hbm.at[idx_v], rows_v, sem).wait()
        pltpu.sync_copy(rows_v, out_hbm.at[pl.ds(base, 128)])


BLK = 1024
GRID = NP // BLK


def _t1_body(degA_ref, degB_ref, x_ref, r_ref, xt_ref):
    deg = jnp.max(degA_ref[...] + degB_ref[...], axis=1, keepdims=True)
    r = lax.rsqrt(jnp.maximum(deg, 1.0))
    rb = jnp.broadcast_to(r, (BLK, D))
    r_ref[...] = rb
    xt_ref[...] = x_ref[...] * rb


def _t1(degA, degB, x_p):
    return pl.pallas_call(
        _t1_body,
        grid=(GRID,),
        in_specs=[pl.BlockSpec((BLK, 16), lambda i: (i, 0)),
                  pl.BlockSpec((BLK, 16), lambda i: (i, 0)),
                  pl.BlockSpec((BLK, D), lambda i: (i, 0))],
        out_specs=[pl.BlockSpec((BLK, D), lambda i: (i, 0)),
                   pl.BlockSpec((BLK, D), lambda i: (i, 0))],
        out_shape=[jax.ShapeDtypeStruct((NP, D), f32),
                   jax.ShapeDtypeStruct((NP, D), f32)],
    )(degA, degB, x_p)


def _t2_body(aggA_ref, aggB_ref, r_ref, W_ref, b_ref, out_ref):
    r = r_ref[...]
    a = (aggA_ref[...] + aggB_ref[...]) * r
    o = jnp.dot(a, W_ref[...], preferred_element_type=f32) + b_ref[...]
    out_ref[...] = jnp.maximum(o, 0.0) * r


def _t2(aggA, aggB, r, W, b):
    return pl.pallas_call(
        _t2_body,
        grid=(GRID,),
        in_specs=[pl.BlockSpec((BLK, D), lambda i: (i, 0)),
                  pl.BlockSpec((BLK, D), lambda i: (i, 0)),
                  pl.BlockSpec((BLK, D), lambda i: (i, 0)),
                  pl.BlockSpec((D, D), lambda i: (0, 0)),
                  pl.BlockSpec((1, D), lambda i: (0, 0))],
        out_specs=pl.BlockSpec((BLK, D), lambda i: (i, 0)),
        out_shape=jax.ShapeDtypeStruct((NP, D), f32),
    )(aggA, aggB, r, W, b)


def _t3_body(aggA_ref, aggB_ref, r_ref, W_ref, b_ref, h_ref, hsum_ref):
    i = pl.program_id(0)
    a = (aggA_ref[...] + aggB_ref[...]) * r_ref[...]
    h = jnp.dot(a, W_ref[...], preferred_element_type=f32) + b_ref[...]
    h_ref[...] = h
    rows = i * BLK + lax.broadcasted_iota(jnp.int32, (BLK, D), 0)
    hm = jnp.where(rows < N_NODES, h, 0.0)

    @pl.when(i == 0)
    def _():
        hsum_ref[...] = jnp.zeros_like(hsum_ref)

    hsum_ref[...] += jnp.sum(hm, axis=0, keepdims=True)


def _t3(aggA, aggB, r, W, b):
    return pl.pallas_call(
        _t3_body,
        grid=(GRID,),
        in_specs=[pl.BlockSpec((BLK, D), lambda i: (i, 0)),
                  pl.BlockSpec((BLK, D), lambda i: (i, 0)),
                  pl.BlockSpec((BLK, D), lambda i: (i, 0)),
                  pl.BlockSpec((D, D), lambda i: (0, 0)),
                  pl.BlockSpec((1, D), lambda i: (0, 0))],
        out_specs=[pl.BlockSpec((BLK, D), lambda i: (i, 0)),
                   pl.BlockSpec((1, D), lambda i: (0, 0))],
        out_shape=[jax.ShapeDtypeStruct((NP, D), f32),
                   jax.ShapeDtypeStruct((1, D), f32)],
    )(aggA, aggB, r, W, b)


def _t4_body(hl_ref, Wa_ref, Wb_ref, Wc_ref, we_ref, hsum_ref, bs1_ref,
             Ws2_ref, bs2_ref, out_ref):
    graph = hsum_ref[...] * (1.0 / N_NODES)
    const = (jnp.dot(we_ref[...], Wa_ref[...], preferred_element_type=f32)
             + jnp.dot(graph, Wc_ref[...], preferred_element_type=f32)
             + bs1_ref[...])
    z = jnp.dot(hl_ref[...], Wb_ref[...], preferred_element_type=f32)
    pre = jnp.maximum(z + const, 0.0)
    s = jnp.dot(pre, Ws2_ref[...], preferred_element_type=f32) + bs2_ref[...]
    rows = lax.broadcasted_iota(jnp.int32, (LP, 1), 0)
    mask = rows < L_LEAF
    logits = jnp.where(mask, s, jnp.full_like(s, -1e30))
    m = jnp.max(logits)
    p = jnp.where(mask, jnp.exp(logits - m), 0.0)
    out_ref[...] = p / jnp.sum(p)


def _t4(hl, Wa, Wb, Wc, we, hsum, bs1, Ws2, bs2):
    return pl.pallas_call(
        _t4_body,
        grid=(1,),
        in_specs=[pl.BlockSpec((LP, D), lambda i: (0, 0)),
                  pl.BlockSpec((D, D), lambda i: (0, 0)),
                  pl.BlockSpec((D, D), lambda i: (0, 0)),
                  pl.BlockSpec((D, D), lambda i: (0, 0)),
                  pl.BlockSpec((1, D), lambda i: (0, 0)),
                  pl.BlockSpec((1, D), lambda i: (0, 0)),
                  pl.BlockSpec((1, D), lambda i: (0, 0)),
                  pl.BlockSpec((D, 1), lambda i: (0, 0)),
                  pl.BlockSpec((1, 1), lambda i: (0, 0))],
        out_specs=pl.BlockSpec((LP, 1), lambda i: (0, 0)),
        out_shape=jax.ShapeDtypeStruct((LP, 1), f32),
    )(hl, Wa, Wb, Wc, we, hsum, bs1, Ws2, bs2)


def kernel(x, edge_index, leaf_inds, word_emb, W1, b1, W2, b2, Ws1, bs1,
           Ws2, bs2):
    src = edge_index[0]
    dst = edge_index[1]
    pad_e = EP - E_EDGES
    pad_idx = jnp.full((pad_e,), N_NODES, jnp.int32)
    src_p = jnp.concatenate([src, pad_idx])
    dst_p = jnp.concatenate([dst, pad_idx])
    dst2 = dst_p.reshape(EP // 128, 128)
    leaf_p = jnp.concatenate(
        [leaf_inds, jnp.full((LP - L_LEAF,), N_NODES, jnp.int32)])
    x_p = jnp.concatenate([x, jnp.zeros((NP - N_NODES, D), f32)], axis=0)
    zerosND = jnp.zeros((NP, D), f32)
    zeros16 = jnp.zeros((NP, 16), f32)
    ones16 = jnp.ones((128, 16), f32)

    deg2 = _deg_kernel(dst2, zeros16, ones16)
    r, xt = _t1(deg2[:NP], deg2[NP:], x_p)
    agg1 = _segsum_kernel(xt, src_p, dst2, zerosND)
    h1t = _t2(agg1[:NP], agg1[NP:], r, W1, b1.reshape(1, D))
    agg2 = _segsum_kernel(h1t, src_p, dst2, zerosND)
    h, hsum = _t3(agg2[:NP], agg2[NP:], r, W2, b2.reshape(1, D))
    hl = _leaf_gather_kernel(h, leaf_p)
    out = _t4(hl, Ws1[:D], Ws1[D:2 * D], Ws1[2 * D:],
              word_emb.reshape(1, D), hsum, bs1.reshape(1, D),
              Ws2, bs2.reshape(1, 1))
    return out[:L_LEAF]


# speedup vs baseline: 7.2247x; 7.2247x over previous
import functools

import jax
import jax.numpy as jnp
from jax import lax
from jax.experimental import pallas as pl
from jax.experimental.pallas import tpu as pltpu
from jax.experimental.pallas import tpu_sc as plsc

N_NODES = 10000
D = 128
E_EDGES = 320000
L_LEAF = 5000

NC = 2
NS = 16
NW = NC * NS

NP = 10240
EP = 323584
EPT = EP // NW
NBLK = EPT // 128
LP = 8192
LPT = LP // NW
LBLK = LPT // 128
RPS = NP // NS

f32 = jnp.float32


@functools.lru_cache(maxsize=None)
def _sc_kernels():
    mesh = plsc.VectorSubcoreMesh(core_axis_name="c", subcore_axis_name="s",
                                  num_cores=NC, num_subcores=NS)

    @functools.partial(
        pl.kernel,
        out_type=jax.ShapeDtypeStruct((NC * NP, D), f32),
        mesh=mesh,
        scratch_types=[
            pltpu.VMEM((128, D), f32),
            pltpu.VMEM((1, 128), jnp.int32),
            pltpu.VMEM_SHARED((NP, D), f32),
        ],
    )
    def _deg_kernel(dst2_hbm, zeros_hbm, ones_hbm, out_hbm,
                    ones_v, dst_v, acc_sh):
        cid = lax.axis_index("c")
        sid = lax.axis_index("s")
        wid = sid * NC + cid
        pltpu.sync_copy(zeros_hbm.at[pl.ds(sid * RPS, RPS)],
                        acc_sh.at[pl.ds(sid * RPS, RPS)])
        pltpu.sync_copy(ones_hbm, ones_v)
        plsc.subcore_barrier()

        @pl.loop(0, NBLK)
        def _(b):
            pltpu.sync_copy(dst2_hbm.at[pl.ds(wid * NBLK + b, 1)], dst_v)
            pltpu.sync_copy(ones_v, acc_sh.at[dst_v.at[0]], add=True)

        plsc.subcore_barrier()
        pltpu.sync_copy(acc_sh.at[pl.ds(sid * RPS, RPS)],
                        out_hbm.at[pl.ds(cid * NP + sid * RPS, RPS)])

    @functools.partial(
        pl.kernel,
        out_type=jax.ShapeDtypeStruct((NC * NP, D), f32),
        mesh=mesh,
        scratch_types=[
            pltpu.VMEM((128,), jnp.int32),
            pltpu.VMEM((1, 128), jnp.int32),
            pltpu.VMEM((128, D), f32),
            pltpu.VMEM_SHARED((NP, D), f32),
            pltpu.SemaphoreType.DMA,
        ],
    )
    def _segsum_kernel(h_hbm, src_hbm, dst2_hbm, zeros_hbm, out_hbm,
                       src_v, dst_v, rows_v, acc_sh, sem):
        cid = lax.axis_index("c")
        sid = lax.axis_index("s")
        wid = sid * NC + cid
        pltpu.sync_copy(zeros_hbm.at[pl.ds(sid * RPS, RPS)],
                        acc_sh.at[pl.ds(sid * RPS, RPS)])
        plsc.subcore_barrier()

        @pl.loop(0, NBLK)
        def _(b):
            base = wid * EPT + b * 128
            pltpu.sync_copy(src_hbm.at[pl.ds(base, 128)], src_v)
            pltpu.async_copy(h_hbm.at[src_v], rows_v, sem).wait()
            pltpu.sync_copy(dst2_hbm.at[pl.ds(wid * NBLK + b, 1)], dst_v)
            pltpu.sync_copy(rows_v, acc_sh.at[dst_v.at[0]], add=True)

        plsc.subcore_barrier()
        pltpu.sync_copy(acc_sh.at[pl.ds(sid * RPS, RPS)],
                        out_hbm.at[pl.ds(cid * NP + sid * RPS, RPS)])

    @functools.partial(
        pl.kernel,
        out_type=jax.ShapeDtypeStruct((LP, D), f32),
        mesh=mesh,
        scratch_types=[
            pltpu.VMEM((128,), jnp.int32),
            pltpu.VMEM((128, D), f32),
            pltpu.SemaphoreType.DMA,
        ],
    )
    def _leaf_gather_kernel(h_hbm, leaf_hbm, out_hbm, idx_v, rows_v, sem):
        cid = lax.axis_index("c")
        sid = lax.axis_index("s")
        wid = sid * NC + cid

        @pl.loop(0, LBLK)
        def _(b):
            base = wid * LPT + b * 128
            pltpu.sync_copy(leaf_hbm.at[pl.ds(base, 128)], idx_v)
            pltpu.async_copy(h_hbm.at[idx_v], rows_v, sem).wait()
            pltpu.sync_copy(rows_v, out_hbm.at[pl.ds(base, 128)])

    return _deg_kernel, _segsum_kernel, _leaf_gather_kernel


BLK = 1024
GRID = NP // BLK


def _t1_body(degA_ref, degB_ref, x_ref, r_ref, xt_ref):
    deg = jnp.max(degA_ref[...] + degB_ref[...], axis=1, keepdims=True)
    r = lax.rsqrt(jnp.maximum(deg, 1.0))
    rb = jnp.broadcast_to(r, (BLK, D))
    r_ref[...] = rb
    xt_ref[...] = x_ref[...] * rb


def _t1(degA, degB, x_p):
    return pl.pallas_call(
        _t1_body,
        grid=(GRID,),
        in_specs=[pl.BlockSpec((BLK, D), lambda i: (i, 0)),
                  pl.BlockSpec((BLK, D), lambda i: (i, 0)),
                  pl.BlockSpec((BLK, D), lambda i: (i, 0))],
        out_specs=[pl.BlockSpec((BLK, D), lambda i: (i, 0)),
                   pl.BlockSpec((BLK, D), lambda i: (i, 0))],
        out_shape=[jax.ShapeDtypeStruct((NP, D), f32),
                   jax.ShapeDtypeStruct((NP, D), f32)],
    )(degA, degB, x_p)


def _t2_body(aggA_ref, aggB_ref, r_ref, W_ref, b_ref, out_ref):
    r = r_ref[...]
    a = (aggA_ref[...] + aggB_ref[...]) * r
    o = jnp.dot(a, W_ref[...], preferred_element_type=f32) + b_ref[...]
    out_ref[...] = jnp.maximum(o, 0.0) * r


def _t2(aggA, aggB, r, W, b):
    return pl.pallas_call(
        _t2_body,
        grid=(GRID,),
        in_specs=[pl.BlockSpec((BLK, D), lambda i: (i, 0)),
                  pl.BlockSpec((BLK, D), lambda i: (i, 0)),
                  pl.BlockSpec((BLK, D), lambda i: (i, 0)),
                  pl.BlockSpec((D, D), lambda i: (0, 0)),
                  pl.BlockSpec((1, D), lambda i: (0, 0))],
        out_specs=pl.BlockSpec((BLK, D), lambda i: (i, 0)),
        out_shape=jax.ShapeDtypeStruct((NP, D), f32),
    )(aggA, aggB, r, W, b)


def _t3_body(aggA_ref, aggB_ref, r_ref, W_ref, b_ref, h_ref, hsum_ref):
    i = pl.program_id(0)
    a = (aggA_ref[...] + aggB_ref[...]) * r_ref[...]
    h = jnp.dot(a, W_ref[...], preferred_element_type=f32) + b_ref[...]
    h_ref[...] = h
    rows = i * BLK + lax.broadcasted_iota(jnp.int32, (BLK, D), 0)
    hm = jnp.where(rows < N_NODES, h, 0.0)

    @pl.when(i == 0)
    def _():
        hsum_ref[...] = jnp.zeros_like(hsum_ref)

    hsum_ref[...] += jnp.sum(hm, axis=0, keepdims=True)


def _t3(aggA, aggB, r, W, b):
    return pl.pallas_call(
        _t3_body,
        grid=(GRID,),
        in_specs=[pl.BlockSpec((BLK, D), lambda i: (i, 0)),
                  pl.BlockSpec((BLK, D), lambda i: (i, 0)),
                  pl.BlockSpec((BLK, D), lambda i: (i, 0)),
                  pl.BlockSpec((D, D), lambda i: (0, 0)),
                  pl.BlockSpec((1, D), lambda i: (0, 0))],
        out_specs=[pl.BlockSpec((BLK, D), lambda i: (i, 0)),
                   pl.BlockSpec((1, D), lambda i: (0, 0))],
        out_shape=[jax.ShapeDtypeStruct((NP, D), f32),
                   jax.ShapeDtypeStruct((1, D), f32)],
    )(aggA, aggB, r, W, b)


def _t4_body(hl_ref, Wa_ref, Wb_ref, Wc_ref, we_ref, hsum_ref, bs1_ref,
             Ws2_ref, bs2_ref, out_ref):
    graph = hsum_ref[...] * (1.0 / N_NODES)
    const = (jnp.dot(we_ref[...], Wa_ref[...], preferred_element_type=f32)
             + jnp.dot(graph, Wc_ref[...], preferred_element_type=f32)
             + bs1_ref[...])
    z = jnp.dot(hl_ref[...], Wb_ref[...], preferred_element_type=f32)
    pre = jnp.maximum(z + const, 0.0)
    s = jnp.dot(pre, Ws2_ref[...], preferred_element_type=f32) + bs2_ref[...]
    rows = lax.broadcasted_iota(jnp.int32, (LP, 1), 0)
    mask = rows < L_LEAF
    logits = jnp.where(mask, s, jnp.full_like(s, -1e30))
    m = jnp.max(logits)
    p = jnp.where(mask, jnp.exp(logits - m), 0.0)
    out_ref[...] = p / jnp.sum(p)


def _t4(hl, Wa, Wb, Wc, we, hsum, bs1, Ws2, bs2):
    return pl.pallas_call(
        _t4_body,
        grid=(1,),
        in_specs=[pl.BlockSpec((LP, D), lambda i: (0, 0)),
                  pl.BlockSpec((D, D), lambda i: (0, 0)),
                  pl.BlockSpec((D, D), lambda i: (0, 0)),
                  pl.BlockSpec((D, D), lambda i: (0, 0)),
                  pl.BlockSpec((1, D), lambda i: (0, 0)),
                  pl.BlockSpec((1, D), lambda i: (0, 0)),
                  pl.BlockSpec((1, D), lambda i: (0, 0)),
                  pl.BlockSpec((D, 1), lambda i: (0, 0)),
                  pl.BlockSpec((1, 1), lambda i: (0, 0))],
        out_specs=pl.BlockSpec((LP, 1), lambda i: (0, 0)),
        out_shape=jax.ShapeDtypeStruct((LP, 1), f32),
    )(hl, Wa, Wb, Wc, we, hsum, bs1, Ws2, bs2)


def kernel(x, edge_index, leaf_inds, word_emb, W1, b1, W2, b2, Ws1, bs1,
           Ws2, bs2):
    src = edge_index[0]
    dst = edge_index[1]
    pad_e = EP - E_EDGES
    pad_idx = jnp.full((pad_e,), N_NODES, jnp.int32)
    src_p = jnp.concatenate([src, pad_idx])
    dst_p = jnp.concatenate([dst, pad_idx])
    dst2 = dst_p.reshape(EP // 128, 128)
    leaf_p = jnp.concatenate(
        [leaf_inds, jnp.full((LP - L_LEAF,), N_NODES, jnp.int32)])
    x_p = jnp.concatenate([x, jnp.zeros((NP - N_NODES, D), f32)], axis=0)
    zerosND = jnp.zeros((NP, D), f32)
    onesND = jnp.ones((128, D), f32)

    deg_k, segsum_k, leaf_k = _sc_kernels()
    deg2 = deg_k(dst2, zerosND, onesND)
    r, xt = _t1(deg2[:NP], deg2[NP:], x_p)
    agg1 = segsum_k(xt, src_p, dst2, zerosND)
    h1t = _t2(agg1[:NP], agg1[NP:], r, W1, b1.reshape(1, D))
    agg2 = segsum_k(h1t, src_p, dst2, zerosND)
    h, hsum = _t3(agg2[:NP], agg2[NP:], r, W2, b2.reshape(1, D))
    hl = leaf_k(h, leaf_p)
    out = _t4(hl, Ws1[:D], Ws1[D:2 * D], Ws1[2 * D:],
              word_emb.reshape(1, D), hsum, bs1.reshape(1, D),
              Ws2, bs2.reshape(1, 1))
    return out[:L_LEAF]
